# baseline (speedup 1.0000x reference)
import functools

import jax
import jax.numpy as jnp
from jax import lax
from jax.experimental import pallas as pl
from jax.experimental.pallas import tpu as pltpu
from jax.experimental.pallas import tpu_sc as plsc

L = 2048
NB = 4
ROWS = 2 * NB * L
NW = 32
SEG = ROWS // (2 * NW)
CHUNK = 16
LANES = 16


def _sc_call(seq, tab):
    mesh = plsc.VectorSubcoreMesh(core_axis_name="c", subcore_axis_name="s")

    @functools.partial(
        pl.kernel,
        mesh=mesh,
        out_type=jax.ShapeDtypeStruct((ROWS, L), jnp.float32),
        scratch_types=[
            pltpu.VMEM((L,), jnp.int32),
            pltpu.VMEM((CHUNK, L), jnp.float32),
            pltpu.SemaphoreType.DMA,
            pltpu.SemaphoreType.DMA,
        ],
    )
    def k(seq_hbm, tab_hbm, out_hbm, seq_v, pat, sem_b, sem_t):
        nc = 2
        wid = lax.axis_index("s") * nc + lax.axis_index("c")
        sym = wid // (NW // NB)
        i0 = wid * SEG - sym * L
        top_base = wid * SEG
        bot_base = NB * L + wid * SEG

        pltpu.sync_copy(seq_hbm, seq_v)

        def pbuild(j, _):
            v = jnp.where(
                seq_v[pl.ds(j * LANES, LANES)] == sym, 1.0, 0.0
            ).astype(jnp.float32)
            for kk in range(CHUNK):
                pat[kk, pl.ds(j * LANES, LANES)] = v
            return 0
        lax.fori_loop(0, L // LANES, pbuild, 0)

        def db(m, _):
            pltpu.async_copy(
                pat, out_hbm.at[pl.ds(bot_base + m * CHUNK, CHUNK)], sem_b)
            return 0
        lax.fori_loop(0, SEG // CHUNK, db, 0)

        def dt(mb, _):
            v16 = seq_v[pl.ds(i0 + mb * LANES, LANES)]
            bvec = jnp.where(v16 == sym, 2 * wid + 1, 2 * wid).astype(
                jnp.int32)
            for kk in range(LANES):
                pltpu.async_copy(
                    tab_hbm.at[pl.ds(bvec[kk], 1)],
                    out_hbm.at[pl.ds(top_base + mb * LANES + kk, 1)], sem_t)
            return 0
        lax.fori_loop(0, SEG // LANES, dt, 0)

        def wt(m, _):
            pltpu.make_async_copy(
                tab_hbm.at[pl.ds(0, 1)],
                out_hbm.at[pl.ds(top_base, 1)], sem_t).wait()
            return 0
        lax.fori_loop(0, SEG, wt, 0)

        def wb(m, _):
            pltpu.make_async_copy(
                pat, out_hbm.at[pl.ds(bot_base, CHUNK)], sem_b).wait()
            return 0
        lax.fori_loop(0, SEG // CHUNK, wb, 0)

    return k(seq, tab)


def kernel(sequence):
    seq = sequence.astype(jnp.int32)
    tab = jnp.tile(
        jnp.stack([jnp.zeros((L,), jnp.float32),
                   jnp.ones((L,), jnp.float32)]), (NW, 1))
    out = _sc_call(seq, tab)
    return out.reshape(2 * NB, L, L)


# speedup vs baseline: 32.7528x; 32.7528x over previous
import functools

import jax
import jax.numpy as jnp
from jax import lax
from jax.experimental import pallas as pl
from jax.experimental.pallas import tpu as pltpu
from jax.experimental.pallas import tpu_sc as plsc

L = 2048
NB = 4
ROWS = 2 * NB * L
NW = 32
RPW = ROWS // NW
CHUNK = 16
NCH = RPW // CHUNK
LANES = 16


def _sc_call(seq, seq_rep):
    mesh = plsc.VectorSubcoreMesh(core_axis_name="c", subcore_axis_name="s")

    @functools.partial(
        pl.kernel,
        mesh=mesh,
        out_type=jax.ShapeDtypeStruct((ROWS, L), jnp.float32),
        scratch_types=[
            pltpu.VMEM((L,), jnp.int32),
            pltpu.VMEM((RPW * LANES,), jnp.int32),
            pltpu.VMEM((CHUNK, L), jnp.float32),
            pltpu.VMEM((CHUNK, L), jnp.float32),
            pltpu.SemaphoreType.DMA,
            pltpu.SemaphoreType.DMA,
        ],
    )
    def k(seq_hbm, rep_hbm, out_hbm, seq_v, rep_v, buf_a, buf_b,
          sem_a, sem_b):
        nc = 2
        wid = lax.axis_index("s") * nc + lax.axis_index("c")
        base = wid * RPW
        ch = wid // (L // RPW)
        i0 = base - ch * L
        is_top = ch < NB

        @pl.when(jnp.logical_not(is_top))
        def _bot():
            pltpu.sync_copy(seq_hbm, seq_v)

            def jb(j, _):
                v = jnp.where(
                    seq_v[pl.ds(j * LANES, LANES)] == (ch - NB), 1.0, 0.0
                ).astype(jnp.float32)
                for kk in range(CHUNK):
                    buf_a[kk, pl.ds(j * LANES, LANES)] = v
                return 0
            lax.fori_loop(0, L // LANES, jb, 0)

            def db(m, _):
                pltpu.async_copy(
                    buf_a, out_hbm.at[pl.ds(base + m * CHUNK, CHUNK)], sem_a)
                return 0
            lax.fori_loop(0, NCH, db, 0)

            def dw(m, _):
                pltpu.make_async_copy(
                    buf_a, out_hbm.at[pl.ds(base, CHUNK)], sem_a).wait()
                return 0
            lax.fori_loop(0, NCH, dw, 0)

        @pl.when(is_top)
        def _top():
            pltpu.sync_copy(rep_hbm.at[pl.ds(i0 * LANES, RPW * LANES)], rep_v)

            def one_chunk(m, buf, sem, do_wait):
                rows = [
                    jnp.where(
                        rep_v[pl.ds((m * CHUNK + kk) * LANES, LANES)] == ch,
                        1.0, 0.0).astype(jnp.float32)
                    for kk in range(CHUNK)
                ]

                @pl.when(do_wait)
                def _w():
                    pltpu.make_async_copy(
                        buf, out_hbm.at[pl.ds(base, CHUNK)], sem).wait()

                def jb(j, _):
                    for kk in range(CHUNK):
                        buf[kk, pl.ds(j * LANES, LANES)] = rows[kk]
                    return 0
                lax.fori_loop(0, L // LANES, jb, 0)

                pltpu.async_copy(
                    buf, out_hbm.at[pl.ds(base + m * CHUNK, CHUNK)], sem)

            def tb(t, _):
                one_chunk(2 * t, buf_a, sem_a, t > 0)
                one_chunk(2 * t + 1, buf_b, sem_b, t > 0)
                return 0
            lax.fori_loop(0, NCH // 2, tb, 0)
            pltpu.make_async_copy(
                buf_a, out_hbm.at[pl.ds(base, CHUNK)], sem_a).wait()
            pltpu.make_async_copy(
                buf_b, out_hbm.at[pl.ds(base, CHUNK)], sem_b).wait()

    return k(seq, seq_rep)


def kernel(sequence):
    seq = sequence.astype(jnp.int32)
    seq_rep = jnp.repeat(seq, LANES)
    out = _sc_call(seq, seq_rep)
    return out.reshape(2 * NB, L, L)
